# baseline (speedup 1.0000x reference)
import functools

import jax
import jax.numpy as jnp
from jax import lax
from jax.experimental import pallas as pl
from jax.experimental.pallas import tpu as pltpu
from jax.experimental.pallas import tpu_sc as plsc

N = 100000
NPAD = 102400
CH = 2048
NCHUNK = 100
EPW = CH * NCHUNK
EPAD = 32 * EPW
ROWS = NPAD // 128

_mesh = plsc.VectorSubcoreMesh(core_axis_name="c", subcore_axis_name="s")


def _aggregate_body(w_hbm, ai_hbm, bi_hbm, dump_hbm,
                    aidx0, aidx1, bidx0, bidx1, msgs0, msgs1, aggr_v,
                    wsh, sem_a, sem_b, sem_g):
    cid = lax.axis_index("c")
    sid = lax.axis_index("s")
    wid = cid * 16 + sid
    base = wid * EPW

    @pl.when(sid == 0)
    def _():
        pltpu.sync_copy(w_hbm, wsh)

    neg_inf = jnp.full((16,), -jnp.inf, jnp.float32)

    def init_body(i, _):
        aggr_v[pl.ds(i * 16, 16)] = neg_inf
        return 0

    lax.fori_loop(0, NPAD // 16, init_body, 0)
    plsc.subcore_barrier()

    aidx = (aidx0, aidx1)
    bidx = (bidx0, bidx1)
    msgs = (msgs0, msgs1)

    def fire_idx(j, p):
        pltpu.async_copy(ai_hbm.at[pl.ds(base + j * CH, CH)], aidx[p], sem_a)
        pltpu.async_copy(bi_hbm.at[pl.ds(base + j * CH, CH)], bidx[p], sem_b)

    def fire_gathers(p):
        for t in range(CH // 128):
            pltpu.async_copy(wsh.at[aidx[p].at[pl.ds(t * 128, 128)]],
                             msgs[p].at[pl.ds(t * 128, 128)], sem_g)

    def wait_a(p):
        pltpu.make_async_copy(ai_hbm.at[pl.ds(0, CH)], aidx[p], sem_a).wait()

    def wait_b(p):
        pltpu.make_async_copy(bi_hbm.at[pl.ds(0, CH)], bidx[p], sem_b).wait()

    def drain_g(p):
        pltpu.make_async_copy(w_hbm.at[pl.ds(0, CH)], msgs[p], sem_g).wait()

    def rmw(p):
        bv = bidx[p]
        mv = msgs[p]

        @plsc.parallel_loop(0, CH // 16, unroll=4)
        def _main(v):
            o = v * 16
            idx = bv[pl.ds(o, 16)]
            msg = mv[pl.ds(o, 16)]
            cur = plsc.load_gather(aggr_v, [idx])
            plsc.store_scatter(aggr_v, [idx], jnp.maximum(cur, msg))

        @plsc.parallel_loop(0, CH // 16, unroll=4,
                            carry=jnp.zeros((16,), jnp.int32))
        def badv(v, bad):
            o = v * 16
            idx = bv[pl.ds(o, 16)]
            msg = mv[pl.ds(o, 16)]
            chk = plsc.load_gather(aggr_v, [idx])
            return jnp.maximum(bad, jnp.where(msg > chk, 1, 0))

        anybad = jnp.max(badv)

        def fix_body(_c):
            def fb(v, bad):
                idx = bv[pl.ds(v * 16, 16)]
                msg = mv[pl.ds(v * 16, 16)]
                cur = plsc.load_gather(aggr_v, [idx])
                plsc.store_scatter(aggr_v, [idx], jnp.maximum(cur, msg),
                                   mask=msg > cur)
                chk = plsc.load_gather(aggr_v, [idx])
                b = jnp.max(jnp.where(msg > chk, 1, 0).astype(jnp.int32))
                return jnp.maximum(bad, b)

            return lax.fori_loop(0, CH // 16, fb, jnp.int32(0))

        lax.while_loop(lambda c: c > 0, fix_body, anybad)

    pltpu.sync_copy(ai_hbm.at[pl.ds(base, CH)], aidx[0])
    pltpu.sync_copy(bi_hbm.at[pl.ds(base, CH)], bidx[0])
    fire_gathers(0)
    fire_idx(1, 1)

    def chunk(j, p, first, last2, last):
        if not last:
            wait_a(1 - p)
            fire_gathers(1 - p)
        if not first:
            wait_b(p)
        drain_g(p)
        rmw(p)
        if not last2 and not last:
            fire_idx(j + 2, p)

    chunk(0, 0, True, False, False)
    chunk(1, 1, False, False, False)

    def pair_body(i, _):
        j = i * 2
        chunk(j, 0, False, False, False)
        chunk(j + 1, 1, False, False, False)
        return 0

    lax.fori_loop(1, NCHUNK // 2 - 1, pair_body, 0)
    chunk(NCHUNK - 2, 0, False, True, False)
    chunk(NCHUNK - 1, 1, False, True, True)

    pltpu.sync_copy(aggr_v, dump_hbm.at[pl.ds(wid * NPAD, NPAD)])


_aggregate = functools.partial(
    pl.kernel,
    mesh=_mesh,
    compiler_params=pltpu.CompilerParams(needs_layout_passes=False),
    out_type=jax.ShapeDtypeStruct((32 * NPAD,), jnp.float32),
    scratch_types=[
        pltpu.VMEM((CH,), jnp.int32),
        pltpu.VMEM((CH,), jnp.int32),
        pltpu.VMEM((CH,), jnp.int32),
        pltpu.VMEM((CH,), jnp.int32),
        pltpu.VMEM((CH,), jnp.float32),
        pltpu.VMEM((CH,), jnp.float32),
        pltpu.VMEM((NPAD,), jnp.float32),
        pltpu.VMEM_SHARED((NPAD,), jnp.float32),
        pltpu.SemaphoreType.DMA,
        pltpu.SemaphoreType.DMA,
        pltpu.SemaphoreType.DMA,
    ],
)(_aggregate_body)


def _merge_body(w_ref, d_ref, c_ref, o_ref, *, neg, set_central):
    w = w_ref[...]
    aggr = d_ref[0]
    for r in range(1, 32):
        aggr = jnp.maximum(aggr, d_ref[r])
    if neg:
        out = jnp.where(aggr <= 0.0, w, jnp.minimum(aggr, w))
    else:
        out = jnp.where(aggr > w, aggr, w)
    if set_central:
        node = (lax.broadcasted_iota(jnp.int32, (ROWS, 128), 0) * 128
                + lax.broadcasted_iota(jnp.int32, (ROWS, 128), 1))
        out = jnp.where(node == c_ref[0], 1.0, out)
    o_ref[...] = out


def _merge(w, dump, central, neg, set_central):
    body = functools.partial(_merge_body, neg=neg, set_central=set_central)
    out = pl.pallas_call(
        body,
        out_shape=jax.ShapeDtypeStruct((ROWS, 128), jnp.float32),
        in_specs=[
            pl.BlockSpec(memory_space=pltpu.VMEM),
            pl.BlockSpec(memory_space=pltpu.VMEM),
            pl.BlockSpec(memory_space=pltpu.SMEM),
        ],
        out_specs=pl.BlockSpec(memory_space=pltpu.VMEM),
    )(w.reshape(ROWS, 128), dump.reshape(32, ROWS, 128), central)
    return out.reshape(NPAD)


def kernel(initial_weight, imp_edge_index, graph_central_node):
    src = imp_edge_index[0]
    dst = imp_edge_index[1]
    pad_idx = jnp.full((EPAD - src.shape[0],), NPAD - 1, jnp.int32)
    srcp = jnp.concatenate([src, pad_idx])
    dstp = jnp.concatenate([dst, pad_idx])
    central = jnp.asarray(graph_central_node, jnp.int32).reshape(1)

    w = jnp.concatenate(
        [initial_weight, jnp.zeros((NPAD - N,), jnp.float32)])
    for r in range(3):
        d = _aggregate(w, srcp, dstp)
        w = _merge(w, d, central, neg=True, set_central=(r == 2))
    for r in range(3):
        d = _aggregate(w, dstp, srcp)
        w = _merge(w, d, central, neg=False, set_central=(r == 2))
    return w[:N]


# speedup vs baseline: 1.2882x; 1.2882x over previous
import functools

import jax
import jax.numpy as jnp
from jax import lax
from jax.experimental import pallas as pl
from jax.experimental.pallas import tpu as pltpu
from jax.experimental.pallas import tpu_sc as plsc

N = 100000
NPAD = 102400
CH = 2048
NCHUNK = 100
EPW = CH * NCHUNK
EPAD = 32 * EPW
ROWS = NPAD // 128

_mesh = plsc.VectorSubcoreMesh(core_axis_name="c", subcore_axis_name="s")


def _aggregate_body(w_hbm, ai_hbm, bi_hbm, dump_hbm,
                    aidx0, aidx1, bidx0, bidx1, msgs0, msgs1, aggr_v,
                    wsh, sem_a, sem_b, sem_g):
    cid = lax.axis_index("c")
    sid = lax.axis_index("s")
    wid = cid * 16 + sid
    base = wid * EPW

    @pl.when(sid == 0)
    def _():
        pltpu.sync_copy(w_hbm, wsh)

    neg_inf = jnp.full((16,), -jnp.inf, jnp.float32)

    def init_body(i, _):
        aggr_v[pl.ds(i * 16, 16)] = neg_inf
        return 0

    lax.fori_loop(0, NPAD // 16, init_body, 0)
    plsc.subcore_barrier()

    aidx = (aidx0, aidx1)
    bidx = (bidx0, bidx1)
    msgs = (msgs0, msgs1)

    def fire_idx(j, p):
        pltpu.async_copy(ai_hbm.at[pl.ds(base + j * CH, CH)], aidx[p], sem_a)
        pltpu.async_copy(bi_hbm.at[pl.ds(base + j * CH, CH)], bidx[p], sem_b)

    def fire_gathers(p):
        for t in range(CH // 128):
            pltpu.async_copy(wsh.at[aidx[p].at[pl.ds(t * 128, 128)]],
                             msgs[p].at[pl.ds(t * 128, 128)], sem_g)

    def wait_a(p):
        pltpu.make_async_copy(ai_hbm.at[pl.ds(0, CH)], aidx[p], sem_a).wait()

    def wait_b(p):
        pltpu.make_async_copy(bi_hbm.at[pl.ds(0, CH)], bidx[p], sem_b).wait()

    def drain_g(p):
        pltpu.make_async_copy(w_hbm.at[pl.ds(0, CH)], msgs[p], sem_g).wait()

    def rmw(p):
        bv = bidx[p]
        mv = msgs[p]

        def vec_body(v, bad):
            for u in range(8):
                o = v * 128 + u * 16
                idx = bv[pl.ds(o, 16)]
                msg = mv[pl.ds(o, 16)]
                cur = plsc.load_gather(aggr_v, [idx])
                plsc.store_scatter(aggr_v, [idx], jnp.maximum(cur, msg))
                chk = plsc.load_gather(aggr_v, [idx])
                bad = jnp.maximum(bad, jnp.where(msg > chk, 1, 0))
            return bad

        badv = lax.fori_loop(0, CH // 128, vec_body,
                             jnp.zeros((16,), jnp.int32))
        anybad = jnp.max(badv)

        def fix_body(_c):
            def fb(v, bad):
                idx = bv[pl.ds(v * 16, 16)]
                msg = mv[pl.ds(v * 16, 16)]
                cur = plsc.load_gather(aggr_v, [idx])
                plsc.store_scatter(aggr_v, [idx], jnp.maximum(cur, msg),
                                   mask=msg > cur)
                chk = plsc.load_gather(aggr_v, [idx])
                b = jnp.max(jnp.where(msg > chk, 1, 0).astype(jnp.int32))
                return jnp.maximum(bad, b)

            return lax.fori_loop(0, CH // 16, fb, jnp.int32(0))

        lax.while_loop(lambda c: c > 0, fix_body, anybad)

    pltpu.sync_copy(ai_hbm.at[pl.ds(base, CH)], aidx[0])
    pltpu.sync_copy(bi_hbm.at[pl.ds(base, CH)], bidx[0])
    fire_gathers(0)
    fire_idx(1, 1)

    def chunk(j, p, first, last2, last):
        if not last:
            wait_a(1 - p)
            fire_gathers(1 - p)
        if not first:
            wait_b(p)
        drain_g(p)
        rmw(p)
        if not last2 and not last:
            fire_idx(j + 2, p)

    chunk(0, 0, True, False, False)
    chunk(1, 1, False, False, False)

    def pair_body(i, _):
        j = i * 2
        chunk(j, 0, False, False, False)
        chunk(j + 1, 1, False, False, False)
        return 0

    lax.fori_loop(1, NCHUNK // 2 - 1, pair_body, 0)
    chunk(NCHUNK - 2, 0, False, True, False)
    chunk(NCHUNK - 1, 1, False, True, True)

    pltpu.sync_copy(aggr_v, dump_hbm.at[pl.ds(wid * NPAD, NPAD)])


_aggregate = functools.partial(
    pl.kernel,
    mesh=_mesh,
    compiler_params=pltpu.CompilerParams(needs_layout_passes=False),
    out_type=jax.ShapeDtypeStruct((32 * NPAD,), jnp.float32),
    scratch_types=[
        pltpu.VMEM((CH,), jnp.int32),
        pltpu.VMEM((CH,), jnp.int32),
        pltpu.VMEM((CH,), jnp.int32),
        pltpu.VMEM((CH,), jnp.int32),
        pltpu.VMEM((CH,), jnp.float32),
        pltpu.VMEM((CH,), jnp.float32),
        pltpu.VMEM((NPAD,), jnp.float32),
        pltpu.VMEM_SHARED((NPAD,), jnp.float32),
        pltpu.SemaphoreType.DMA,
        pltpu.SemaphoreType.DMA,
        pltpu.SemaphoreType.DMA,
    ],
)(_aggregate_body)


def _merge_body(w_ref, d_ref, c_ref, o_ref, *, neg, set_central):
    w = w_ref[...]
    aggr = d_ref[0]
    for r in range(1, 32):
        aggr = jnp.maximum(aggr, d_ref[r])
    if neg:
        out = jnp.where(aggr <= 0.0, w, jnp.minimum(aggr, w))
    else:
        out = jnp.where(aggr > w, aggr, w)
    if set_central:
        node = (lax.broadcasted_iota(jnp.int32, (ROWS, 128), 0) * 128
                + lax.broadcasted_iota(jnp.int32, (ROWS, 128), 1))
        out = jnp.where(node == c_ref[0], 1.0, out)
    o_ref[...] = out


def _merge(w, dump, central, neg, set_central):
    body = functools.partial(_merge_body, neg=neg, set_central=set_central)
    out = pl.pallas_call(
        body,
        out_shape=jax.ShapeDtypeStruct((ROWS, 128), jnp.float32),
        in_specs=[
            pl.BlockSpec(memory_space=pltpu.VMEM),
            pl.BlockSpec(memory_space=pltpu.VMEM),
            pl.BlockSpec(memory_space=pltpu.SMEM),
        ],
        out_specs=pl.BlockSpec(memory_space=pltpu.VMEM),
    )(w.reshape(ROWS, 128), dump.reshape(32, ROWS, 128), central)
    return out.reshape(NPAD)


def kernel(initial_weight, imp_edge_index, graph_central_node):
    src = imp_edge_index[0]
    dst = imp_edge_index[1]
    pad_idx = jnp.full((EPAD - src.shape[0],), NPAD - 1, jnp.int32)
    srcp = jnp.concatenate([src, pad_idx])
    dstp = jnp.concatenate([dst, pad_idx])
    central = jnp.asarray(graph_central_node, jnp.int32).reshape(1)

    w = jnp.concatenate(
        [initial_weight, jnp.zeros((NPAD - N,), jnp.float32)])
    for r in range(3):
        d = _aggregate(w, srcp, dstp)
        w = _merge(w, d, central, neg=True, set_central=(r == 2))
    for r in range(3):
        d = _aggregate(w, dstp, srcp)
        w = _merge(w, d, central, neg=False, set_central=(r == 2))
    return w[:N]


# speedup vs baseline: 1.3995x; 1.0864x over previous
import functools

import jax
import jax.numpy as jnp
from jax import lax
from jax.experimental import pallas as pl
from jax.experimental.pallas import tpu as pltpu
from jax.experimental.pallas import tpu_sc as plsc

N = 100000
NPAD = 102400
CH = 2048
NCHUNK = 100
EPW = CH * NCHUNK
EPAD = 32 * EPW
ROWS = NPAD // 128

_mesh = plsc.VectorSubcoreMesh(core_axis_name="c", subcore_axis_name="s")


def _aggregate_body(w_hbm, ai_hbm, bi_hbm, dump_hbm,
                    aidx0, aidx1, bidx0, bidx1, msgs0, msgs1, aggr_v,
                    wsh, sem_a, sem_b, sem_g):
    cid = lax.axis_index("c")
    sid = lax.axis_index("s")
    wid = cid * 16 + sid
    base = wid * EPW

    @pl.when(sid == 0)
    def _():
        pltpu.sync_copy(w_hbm, wsh)

    neg_inf = jnp.full((16,), -jnp.inf, jnp.float32)

    def init_body(i, _):
        for u in range(8):
            aggr_v[pl.ds(i * 128 + u * 16, 16)] = neg_inf
        return 0

    lax.fori_loop(0, NPAD // 128, init_body, 0)
    plsc.subcore_barrier()

    aidx = (aidx0, aidx1)
    bidx = (bidx0, bidx1)
    msgs = (msgs0, msgs1)

    def fire_idx(j, p):
        pltpu.async_copy(ai_hbm.at[pl.ds(base + j * CH, CH)], aidx[p], sem_a)
        pltpu.async_copy(bi_hbm.at[pl.ds(base + j * CH, CH)], bidx[p], sem_b)

    def fire_gathers(p):
        for t in range(CH // 128):
            pltpu.async_copy(wsh.at[aidx[p].at[pl.ds(t * 128, 128)]],
                             msgs[p].at[pl.ds(t * 128, 128)], sem_g)

    def wait_a(p):
        pltpu.make_async_copy(ai_hbm.at[pl.ds(0, CH)], aidx[p], sem_a).wait()

    def wait_b(p):
        pltpu.make_async_copy(bi_hbm.at[pl.ds(0, CH)], bidx[p], sem_b).wait()

    def drain_g(p):
        pltpu.make_async_copy(w_hbm.at[pl.ds(0, CH)], msgs[p], sem_g).wait()

    def rmw(p):
        bv = bidx[p]
        mv = msgs[p]

        def vec_body(v, bad):
            for u in range(8):
                o = v * 128 + u * 16
                idx = bv[pl.ds(o, 16)]
                msg = mv[pl.ds(o, 16)]
                cur = plsc.load_gather(aggr_v, [idx])
                plsc.store_scatter(aggr_v, [idx], jnp.maximum(cur, msg))
                chk = plsc.load_gather(aggr_v, [idx])
                bad = jnp.maximum(bad, jnp.where(msg > chk, 1, 0))
            return bad

        badv = lax.fori_loop(0, CH // 128, vec_body,
                             jnp.zeros((16,), jnp.int32))
        anybad = jnp.max(badv)

        def fix_body(_c):
            def fb(v, bad):
                idx = bv[pl.ds(v * 16, 16)]
                msg = mv[pl.ds(v * 16, 16)]
                cur = plsc.load_gather(aggr_v, [idx])
                plsc.store_scatter(aggr_v, [idx], jnp.maximum(cur, msg),
                                   mask=msg > cur)
                chk = plsc.load_gather(aggr_v, [idx])
                b = jnp.max(jnp.where(msg > chk, 1, 0).astype(jnp.int32))
                return jnp.maximum(bad, b)

            return lax.fori_loop(0, CH // 16, fb, jnp.int32(0))

        lax.while_loop(lambda c: c > 0, fix_body, anybad)

    pltpu.sync_copy(ai_hbm.at[pl.ds(base, CH)], aidx[0])
    pltpu.sync_copy(bi_hbm.at[pl.ds(base, CH)], bidx[0])
    fire_gathers(0)
    fire_idx(1, 1)

    def chunk(j, p, first, last2, last):
        if not last:
            wait_a(1 - p)
            fire_gathers(1 - p)
        if not first:
            wait_b(p)
        drain_g(p)
        rmw(p)
        if not last2 and not last:
            fire_idx(j + 2, p)

    chunk(0, 0, True, False, False)
    chunk(1, 1, False, False, False)

    def pair_body(i, _):
        j = i * 2
        chunk(j, 0, False, False, False)
        chunk(j + 1, 1, False, False, False)
        return 0

    lax.fori_loop(1, NCHUNK // 2 - 1, pair_body, 0)
    chunk(NCHUNK - 2, 0, False, True, False)
    chunk(NCHUNK - 1, 1, False, True, True)

    pltpu.sync_copy(aggr_v, dump_hbm.at[pl.ds(wid * NPAD, NPAD)])


_aggregate = functools.partial(
    pl.kernel,
    mesh=_mesh,
    compiler_params=pltpu.CompilerParams(needs_layout_passes=False),
    out_type=jax.ShapeDtypeStruct((32 * NPAD,), jnp.float32),
    scratch_types=[
        pltpu.VMEM((CH,), jnp.int32),
        pltpu.VMEM((CH,), jnp.int32),
        pltpu.VMEM((CH,), jnp.int32),
        pltpu.VMEM((CH,), jnp.int32),
        pltpu.VMEM((CH,), jnp.float32),
        pltpu.VMEM((CH,), jnp.float32),
        pltpu.VMEM((NPAD,), jnp.float32),
        pltpu.VMEM_SHARED((NPAD,), jnp.float32),
        pltpu.SemaphoreType.DMA,
        pltpu.SemaphoreType.DMA,
        pltpu.SemaphoreType.DMA,
    ],
)(_aggregate_body)


def _merge_body(w_ref, d_ref, c_ref, o_ref, *, neg, set_central):
    w = w_ref[...]
    aggr = d_ref[0]
    for r in range(1, 32):
        aggr = jnp.maximum(aggr, d_ref[r])
    if neg:
        out = jnp.where(aggr <= 0.0, w, jnp.minimum(aggr, w))
    else:
        out = jnp.where(aggr > w, aggr, w)
    if set_central:
        node = (lax.broadcasted_iota(jnp.int32, (ROWS, 128), 0) * 128
                + lax.broadcasted_iota(jnp.int32, (ROWS, 128), 1))
        out = jnp.where(node == c_ref[0], 1.0, out)
    o_ref[...] = out


def _merge(w, dump, central, neg, set_central):
    body = functools.partial(_merge_body, neg=neg, set_central=set_central)
    out = pl.pallas_call(
        body,
        out_shape=jax.ShapeDtypeStruct((ROWS, 128), jnp.float32),
        in_specs=[
            pl.BlockSpec(memory_space=pltpu.VMEM),
            pl.BlockSpec(memory_space=pltpu.VMEM),
            pl.BlockSpec(memory_space=pltpu.SMEM),
        ],
        out_specs=pl.BlockSpec(memory_space=pltpu.VMEM),
    )(w.reshape(ROWS, 128), dump.reshape(32, ROWS, 128), central)
    return out.reshape(NPAD)


def kernel(initial_weight, imp_edge_index, graph_central_node):
    src = imp_edge_index[0]
    dst = imp_edge_index[1]
    pad_idx = jnp.full((EPAD - src.shape[0],), NPAD - 1, jnp.int32)
    srcp = jnp.concatenate([src, pad_idx])
    dstp = jnp.concatenate([dst, pad_idx])
    central = jnp.asarray(graph_central_node, jnp.int32).reshape(1)

    w = jnp.concatenate(
        [initial_weight, jnp.zeros((NPAD - N,), jnp.float32)])
    for r in range(3):
        d = _aggregate(w, srcp, dstp)
        w = _merge(w, d, central, neg=True, set_central=(r == 2))
    for r in range(3):
        d = _aggregate(w, dstp, srcp)
        w = _merge(w, d, central, neg=False, set_central=(r == 2))
    return w[:N]
